# baseline (speedup 1.0000x reference)
import functools

import jax
import jax.numpy as jnp
from jax import lax
from jax.experimental import pallas as pl
from jax.experimental.pallas import tpu as pltpu
from jax.experimental.pallas import tpu_sc as plsc

NC = 2
NS = 16
NW = NC * NS
LANES = 16

CB = 4


def _sc_bag_sums(x, emb, B, L, D):
    BPT = B // NW
    NCHUNK = BPT // CB
    NG = D // LANES
    n_full = L // 128
    rem = L - n_full * 128
    pieces = [(i * 128, 128) for i in range(n_full)]
    if rem:
        pieces.append((n_full * 128, rem))

    mesh = plsc.VectorSubcoreMesh(core_axis_name="c", subcore_axis_name="s")

    @functools.partial(
        pl.kernel,
        out_type=jax.ShapeDtypeStruct((B, D), jnp.float32),
        mesh=mesh,
        compiler_params=pltpu.CompilerParams(use_tc_tiling_on_sc=False),
        scratch_types=[
            pltpu.VMEM((CB, L), jnp.int32),
            pltpu.VMEM((CB, L), jnp.int32),
            pltpu.VMEM((CB * L, D), jnp.float32),
            pltpu.VMEM((CB * L, D), jnp.float32),
            pltpu.VMEM((BPT, D), jnp.float32),
            pltpu.SemaphoreType.DMA,
            pltpu.SemaphoreType.DMA,
            pltpu.SemaphoreType.DMA,
            pltpu.SemaphoreType.DMA,
        ],
    )
    def body(x_hbm, emb_hbm, out_hbm, idx0, idx1, rows0, rows1, out_v,
             semg0, semg1, semi0, semi1):
        cid = lax.axis_index("c")
        sid = lax.axis_index("s")
        wid = sid * NC + cid
        base = wid * BPT
        bufs = ((idx0, rows0, semg0, semi0), (idx1, rows1, semg1, semi1))

        def fire_idx(c, buf):
            idx_v, _, _, semi = buf
            pltpu.async_copy(x_hbm.at[pl.ds(base + c * CB, CB)], idx_v, semi)

        def wait_idx(buf):
            idx_v, _, _, semi = buf
            pltpu.make_async_copy(x_hbm.at[pl.ds(base, CB)], idx_v, semi).wait()

        def gather_descs(buf, make_only):
            idx_v, rows_v, semg, _ = buf
            mk = pltpu.make_async_copy if make_only else (
                lambda s, d, sm: pltpu.async_copy(s, d, sm))
            return [
                mk(emb_hbm.at[idx_v.at[j, pl.ds(off, n)]],
                   rows_v.at[pl.ds(j * L + off, n)], semg)
                for j in range(CB) for (off, n) in pieces
            ]

        def reduce_chunk(c, buf):
            rows_v = buf[1]
            for j in range(CB):
                def red(i, carry):
                    r = j * L + i
                    return tuple(
                        carry[g] + rows_v[r, pl.ds(g * LANES, LANES)]
                        for g in range(NG)
                    )
                zero = jnp.zeros((LANES,), jnp.float32)
                acc = lax.fori_loop(0, L, red, (zero,) * NG, unroll=8)
                row = c * CB + j
                for g in range(NG):
                    out_v[row, pl.ds(g * LANES, LANES)] = acc[g]

        fire_idx(0, bufs[0])
        fire_idx(1, bufs[1])
        wait_idx(bufs[0])
        gather_descs(bufs[0], make_only=False)

        @pl.loop(0, NCHUNK // 2)
        def pipe(t):
            for b_ in (0, 1):
                c = t * 2 + b_
                cur, nxt = bufs[b_], bufs[1 - b_]

                @pl.when(c + 1 < NCHUNK)
                def _():
                    wait_idx(nxt)
                    gather_descs(nxt, make_only=False)

                for d in gather_descs(cur, make_only=True):
                    d.wait()

                @pl.when(c + 2 < NCHUNK)
                def _():
                    fire_idx(c + 2, cur)

                reduce_chunk(c, cur)

        pltpu.sync_copy(out_v, out_hbm.at[pl.ds(base, BPT)])

    return body(x, emb)


def _tc_head_body(sa_ref, sb_ref, l_ref, w_ref, b_ref, o_ref):
    denom = jnp.maximum(l_ref[...].astype(jnp.float32), 1.0)
    d2 = sa_ref.shape[1]
    wa = w_ref[:, :d2]
    wb = w_ref[:, d2:]
    acc = lax.dot_general(sa_ref[...] / denom, wa,
                          (((1,), (1,)), ((), ())),
                          preferred_element_type=jnp.float32)
    acc += lax.dot_general(sb_ref[...] / denom, wb,
                           (((1,), (1,)), ((), ())),
                           preferred_element_type=jnp.float32)
    o_ref[...] = acc + b_ref[...]


def _tc_head(sums_a, sums_b, lengths, W, b, B, D, C):
    BLK = 2048
    D2 = D // 2
    len2d = lengths.astype(jnp.int32).reshape(B, 1)
    b2d = b.reshape(1, C)
    return pl.pallas_call(
        _tc_head_body,
        grid=(B // BLK,),
        in_specs=[
            pl.BlockSpec((BLK, D2), lambda i: (i, 0)),
            pl.BlockSpec((BLK, D2), lambda i: (i, 0)),
            pl.BlockSpec((BLK, 1), lambda i: (i, 0)),
            pl.BlockSpec((C, D), lambda i: (0, 0)),
            pl.BlockSpec((1, C), lambda i: (0, 0)),
        ],
        out_specs=pl.BlockSpec((BLK, C), lambda i: (i, 0)),
        out_shape=jax.ShapeDtypeStruct((B, C), jnp.float32),
    )(sums_a, sums_b, len2d, W, b2d)


@jax.jit
def kernel(x, lengths, emb, W, b):
    B, L = x.shape
    V, D = emb.shape
    C = W.shape[0]
    x = x.astype(jnp.int32)
    D2 = D // 2
    sums_a = _sc_bag_sums(x, emb[:, :D2], B, L, D2)
    sums_b = _sc_bag_sums(x, emb[:, D2:], B, L, D2)
    return _tc_head(sums_a, sums_b, lengths, W, b, B, D, C)


# speedup vs baseline: 2.0731x; 2.0731x over previous
import functools

import jax
import jax.numpy as jnp
from jax import lax
from jax.experimental import pallas as pl
from jax.experimental.pallas import tpu as pltpu
from jax.experimental.pallas import tpu_sc as plsc

NC = 2
NS = 16
NW = NC * NS
LANES = 16

CB = 4


def _sc_bag_sums(x, emb, B, L, D):
    BPT = B // NW
    NCHUNK = BPT // CB
    n_full = L // 128
    rem = L - n_full * 128
    pieces = [(i * 128, 128) for i in range(n_full)]
    if rem:
        pieces.append((n_full * 128, rem))

    mesh = plsc.VectorSubcoreMesh(core_axis_name="c", subcore_axis_name="s")

    @functools.partial(
        pl.kernel,
        out_type=jax.ShapeDtypeStruct((B, D), jnp.float32),
        mesh=mesh,
        compiler_params=pltpu.CompilerParams(use_tc_tiling_on_sc=False),
        scratch_types=[
            pltpu.VMEM((CB, L), jnp.int32),
            pltpu.VMEM((CB, L), jnp.int32),
            pltpu.VMEM((CB * L, D), jnp.float32),
            pltpu.VMEM((CB * L, D), jnp.float32),
            pltpu.VMEM((BPT, D), jnp.float32),
            pltpu.SemaphoreType.DMA,
            pltpu.SemaphoreType.DMA,
            pltpu.SemaphoreType.DMA,
            pltpu.SemaphoreType.DMA,
        ],
    )
    def body(x_hbm, emb_hbm, out_hbm, idx0, idx1, rows0, rows1, out_v,
             semg0, semg1, semi0, semi1):
        cid = lax.axis_index("c")
        sid = lax.axis_index("s")
        wid = sid * NC + cid
        base = wid * BPT
        bufs = ((idx0, rows0, semg0, semi0), (idx1, rows1, semg1, semi1))

        def fire_idx(c, buf):
            idx_v, _, _, semi = buf
            pltpu.async_copy(x_hbm.at[pl.ds(base + c * CB, CB)], idx_v, semi)

        def wait_idx(buf):
            idx_v, _, _, semi = buf
            pltpu.make_async_copy(x_hbm.at[pl.ds(base, CB)], idx_v, semi).wait()

        def gather_descs(buf, make_only):
            idx_v, rows_v, semg, _ = buf
            mk = pltpu.make_async_copy if make_only else (
                lambda s, d, sm: pltpu.async_copy(s, d, sm))
            return [
                mk(emb_hbm.at[idx_v.at[j, pl.ds(off, n)]],
                   rows_v.at[pl.ds(j * L + off, n)], semg)
                for j in range(CB) for (off, n) in pieces
            ]

        def reduce_chunk(c, buf):
            rows_v = buf[1]
            for j in range(CB):
                def red(i, carry):
                    a0, a1 = carry
                    r = j * L + i
                    return (a0 + rows_v[r, pl.ds(0, LANES)],
                            a1 + rows_v[r, pl.ds(LANES, LANES)])
                zero = jnp.zeros((LANES,), jnp.float32)
                a0, a1 = lax.fori_loop(0, L, red, (zero, zero), unroll=16)
                row = c * CB + j
                out_v[row, pl.ds(0, LANES)] = a0
                out_v[row, pl.ds(LANES, LANES)] = a1

        fire_idx(0, bufs[0])
        fire_idx(1, bufs[1])
        wait_idx(bufs[0])
        gather_descs(bufs[0], make_only=False)

        @pl.loop(0, NCHUNK // 2)
        def pipe(t):
            for b_ in (0, 1):
                c = t * 2 + b_
                cur, nxt = bufs[b_], bufs[1 - b_]

                @pl.when(c + 1 < NCHUNK)
                def _():
                    wait_idx(nxt)
                    gather_descs(nxt, make_only=False)

                for d in gather_descs(cur, make_only=True):
                    d.wait()

                @pl.when(c + 2 < NCHUNK)
                def _():
                    fire_idx(c + 2, cur)

                reduce_chunk(c, cur)

        pltpu.sync_copy(out_v, out_hbm.at[pl.ds(base, BPT)])

    return body(x, emb)


def _tc_head_body(s_ref, l_ref, w_ref, b_ref, o_ref):
    denom = jnp.maximum(l_ref[...].astype(jnp.float32), 1.0)
    mean = s_ref[...] / denom
    acc = lax.dot_general(mean, w_ref[...],
                          (((1,), (1,)), ((), ())),
                          preferred_element_type=jnp.float32)
    o_ref[...] = acc + b_ref[...]


def _tc_head(sums, lengths, W, b, B, D, C):
    BLK = 2048
    len2d = lengths.astype(jnp.int32).reshape(B, 1)
    b2d = b.reshape(1, C)
    return pl.pallas_call(
        _tc_head_body,
        grid=(B // BLK,),
        in_specs=[
            pl.BlockSpec((BLK, D), lambda i: (i, 0)),
            pl.BlockSpec((BLK, 1), lambda i: (i, 0)),
            pl.BlockSpec((C, D), lambda i: (0, 0)),
            pl.BlockSpec((1, C), lambda i: (0, 0)),
        ],
        out_specs=pl.BlockSpec((BLK, C), lambda i: (i, 0)),
        out_shape=jax.ShapeDtypeStruct((B, C), jnp.float32),
    )(sums, len2d, W, b2d)


@jax.jit
def kernel(x, lengths, emb, W, b):
    B, L = x.shape
    V, D = emb.shape
    C = W.shape[0]
    x = x.astype(jnp.int32)
    sums = _sc_bag_sums(x, emb, B, L, D)
    return _tc_head(sums, lengths, W, b, B, D, C)


# speedup vs baseline: 2.0770x; 1.0018x over previous
import functools

import jax
import jax.numpy as jnp
from jax import lax
from jax.experimental import pallas as pl
from jax.experimental.pallas import tpu as pltpu
from jax.experimental.pallas import tpu_sc as plsc

NC = 2
NS = 16
NW = NC * NS
LANES = 16

CB = 4


def _sc_bag_sums(x, emb, B, L, D):
    BPT = B // NW
    NCHUNK = BPT // CB
    n_full = L // 128
    rem = L - n_full * 128
    pieces = [(i * 128, 128) for i in range(n_full)]
    if rem:
        pieces.append((n_full * 128, rem))

    mesh = plsc.VectorSubcoreMesh(core_axis_name="c", subcore_axis_name="s")

    @functools.partial(
        pl.kernel,
        out_type=jax.ShapeDtypeStruct((B, D), jnp.float32),
        mesh=mesh,
        compiler_params=pltpu.CompilerParams(use_tc_tiling_on_sc=False),
        scratch_types=[
            pltpu.VMEM((CB, L), jnp.int32),
            pltpu.VMEM((CB, L), jnp.int32),
            pltpu.VMEM((CB * L, D), jnp.float32),
            pltpu.VMEM((CB * L, D), jnp.float32),
            pltpu.VMEM((BPT, D), jnp.float32),
            pltpu.SemaphoreType.DMA,
            pltpu.SemaphoreType.DMA,
            pltpu.SemaphoreType.DMA,
            pltpu.SemaphoreType.DMA,
        ],
    )
    def body(x_hbm, emb_hbm, out_hbm, idx0, idx1, rows0, rows1, out_v,
             semg0, semg1, semi0, semi1):
        cid = lax.axis_index("c")
        sid = lax.axis_index("s")
        wid = sid * NC + cid
        base = wid * BPT
        bufs = ((idx0, rows0, semg0, semi0), (idx1, rows1, semg1, semi1))

        def fire_idx(c, buf):
            idx_v, _, _, semi = buf
            pltpu.async_copy(x_hbm.at[pl.ds(base + c * CB, CB)], idx_v, semi)

        def wait_idx(buf):
            idx_v, _, _, semi = buf
            pltpu.make_async_copy(x_hbm.at[pl.ds(base, CB)], idx_v, semi).wait()

        def gather_descs(buf, make_only):
            idx_v, rows_v, semg, _ = buf
            mk = pltpu.make_async_copy if make_only else (
                lambda s, d, sm: pltpu.async_copy(s, d, sm))
            return [
                mk(emb_hbm.at[idx_v.at[j, pl.ds(off, n)]],
                   rows_v.at[pl.ds(j * L + off, n)], semg)
                for j in range(CB) for (off, n) in pieces
            ]

        def reduce_chunk(c, buf):
            rows_v = buf[1]
            for j in range(CB):
                def red(i, carry):
                    a0, a1 = carry
                    r = j * L + i
                    return (a0 + rows_v[r, pl.ds(0, LANES)],
                            a1 + rows_v[r, pl.ds(LANES, LANES)])
                zero = jnp.zeros((LANES,), jnp.float32)
                a0, a1 = lax.fori_loop(0, L, red, (zero, zero), unroll=8)
                row = c * CB + j
                out_v[row, pl.ds(0, LANES)] = a0
                out_v[row, pl.ds(LANES, LANES)] = a1

        fire_idx(0, bufs[0])
        fire_idx(1, bufs[1])
        wait_idx(bufs[0])
        gather_descs(bufs[0], make_only=False)

        @pl.loop(0, NCHUNK // 2)
        def pipe(t):
            for b_ in (0, 1):
                c = t * 2 + b_
                cur, nxt = bufs[b_], bufs[1 - b_]

                @pl.when(c + 1 < NCHUNK)
                def _():
                    wait_idx(nxt)
                    gather_descs(nxt, make_only=False)

                for d in gather_descs(cur, make_only=True):
                    d.wait()

                @pl.when(c + 2 < NCHUNK)
                def _():
                    fire_idx(c + 2, cur)

                reduce_chunk(c, cur)

        pltpu.sync_copy(out_v, out_hbm.at[pl.ds(base, BPT)])

    return body(x, emb)


def _tc_head_body(s_ref, l_ref, w_ref, b_ref, o_ref):
    denom = jnp.maximum(l_ref[...].astype(jnp.float32), 1.0)
    mean = s_ref[...] / denom
    acc = lax.dot_general(mean, w_ref[...],
                          (((1,), (1,)), ((), ())),
                          preferred_element_type=jnp.float32)
    o_ref[...] = acc + b_ref[...]


def _tc_head(sums, lengths, W, b, B, D, C):
    BLK = 2048
    len2d = lengths.astype(jnp.int32).reshape(B, 1)
    b2d = b.reshape(1, C)
    return pl.pallas_call(
        _tc_head_body,
        grid=(B // BLK,),
        in_specs=[
            pl.BlockSpec((BLK, D), lambda i: (i, 0)),
            pl.BlockSpec((BLK, 1), lambda i: (i, 0)),
            pl.BlockSpec((C, D), lambda i: (0, 0)),
            pl.BlockSpec((1, C), lambda i: (0, 0)),
        ],
        out_specs=pl.BlockSpec((BLK, C), lambda i: (i, 0)),
        out_shape=jax.ShapeDtypeStruct((B, C), jnp.float32),
    )(sums, len2d, W, b2d)


@jax.jit
def kernel(x, lengths, emb, W, b):
    B, L = x.shape
    V, D = emb.shape
    C = W.shape[0]
    x = x.astype(jnp.int32)
    sums = _sc_bag_sums(x, emb, B, L, D)
    return _tc_head(sums, lengths, W, b, B, D, C)
